# baseline (speedup 1.0000x reference)
import functools

import jax
import jax.numpy as jnp
from jax import lax
from jax.experimental import pallas as pl
from jax.experimental.pallas import tpu as pltpu
from jax.experimental.pallas import tpu_sc as plsc

_BATCH = 16384
_D = 64
_H1 = 128
_H2 = 64
N_U = 1000000
N_M = 1000000
_CH = 128


def _sc_gather(users_q, movies_q, U2, M2):
    info = plsc.get_sparse_core_info()
    nw = info.num_cores * info.num_subcores
    b_per_w = _BATCH // nw
    n_chunks = b_per_w // _CH

    mesh = plsc.VectorSubcoreMesh(core_axis_name="c", subcore_axis_name="s")

    @functools.partial(
        pl.kernel,
        mesh=mesh,
        out_type=[
            jax.ShapeDtypeStruct((_BATCH, 2 * _D), jnp.float32),
            jax.ShapeDtypeStruct((_BATCH, 2 * _D), jnp.float32),
        ],
        scratch_types=[
            pltpu.VMEM((n_chunks, _CH), jnp.int32),
            pltpu.VMEM((n_chunks, _CH), jnp.int32),
            pltpu.VMEM((b_per_w, 2 * _D), jnp.float32),
            pltpu.SemaphoreType.DMA,
        ],
    )
    def gather_kernel(users_hbm, movies_hbm, u_hbm, m_hbm, ue_hbm, me_hbm,
                      uidx_v, midx_v, rows_v, sem):
        wid = lax.axis_index("s") * info.num_cores + lax.axis_index("c")
        base = wid * b_per_w
        pltpu.sync_copy(users_hbm.at[wid], uidx_v)
        pltpu.sync_copy(movies_hbm.at[wid], midx_v)
        for c in range(n_chunks):
            pltpu.async_copy(u_hbm.at[uidx_v.at[c]],
                             rows_v.at[pl.ds(c * _CH, _CH)], sem)
        pltpu.make_async_copy(
            u_hbm.at[pl.ds(0, b_per_w)], rows_v, sem).wait()
        pltpu.sync_copy(rows_v, ue_hbm.at[pl.ds(base, b_per_w)])
        for c in range(n_chunks):
            pltpu.async_copy(m_hbm.at[midx_v.at[c]],
                             rows_v.at[pl.ds(c * _CH, _CH)], sem)
        pltpu.make_async_copy(
            m_hbm.at[pl.ds(0, b_per_w)], rows_v, sem).wait()
        pltpu.sync_copy(rows_v, me_hbm.at[pl.ds(base, b_per_w)])

    users3 = users_q.reshape(nw, n_chunks, _CH)
    movies3 = movies_q.reshape(nw, n_chunks, _CH)
    return gather_kernel(users3, movies3, U2, M2)


def _repack_body(ulo_ref, uhi_ref, mlo_ref, mhi_ref, u2_ref, m2_ref):
    u2_ref[:, :_D] = ulo_ref[...]
    u2_ref[:, _D:] = uhi_ref[...]
    m2_ref[:, :_D] = mlo_ref[...]
    m2_ref[:, _D:] = mhi_ref[...]


def _repack(U, M):
    br = 2000
    grid = (N_U // 2 // br,)
    half = N_U // 2 // br
    return pl.pallas_call(
        _repack_body,
        grid=grid,
        in_specs=[
            pl.BlockSpec((br, _D), lambda i: (i, 0)),
            pl.BlockSpec((br, _D), lambda i: (i + half, 0)),
            pl.BlockSpec((br, _D), lambda i: (i, 0)),
            pl.BlockSpec((br, _D), lambda i: (i + half, 0)),
        ],
        out_specs=[
            pl.BlockSpec((br, 2 * _D), lambda i: (i, 0)),
            pl.BlockSpec((br, 2 * _D), lambda i: (i, 0)),
        ],
        out_shape=[
            jax.ShapeDtypeStruct((N_U // 2, 2 * _D), jnp.float32),
            jax.ShapeDtypeStruct((N_M // 2, 2 * _D), jnp.float32),
        ],
    )(U, U, M, M)


def _mlp_body(ue_ref, me_ref, pu_ref, pm_ref, w1u_ref, w1m_ref, b1_ref,
              w2_ref, b2_ref, wf_ref, bf_ref, out_ref):
    uL = ue_ref[:, :_D]
    uR = ue_ref[:, _D:]
    mL = me_ref[:, :_D]
    mR = me_ref[:, _D:]
    ue = uL + pu_ref[...] * (uR - uL)
    me = mL + pm_ref[...] * (mR - mL)
    x = jnp.dot(ue, w1u_ref[...], preferred_element_type=jnp.float32)
    x = x + jnp.dot(me, w1m_ref[...], preferred_element_type=jnp.float32)
    x = jnp.maximum(x + b1_ref[...], 0.0)
    x = jnp.dot(x, w2_ref[...], preferred_element_type=jnp.float32)
    x = jnp.maximum(x + b2_ref[...], 0.0)
    x = jnp.dot(x, wf_ref[...], preferred_element_type=jnp.float32)
    out_ref[...] = jax.nn.sigmoid(x + bf_ref[...])


def _mlp(ue, me, pu, pm, W1, b1, W2, b2, Wf, bf):
    w1t = W1.T
    w1u = w1t[:_D]
    w1m = w1t[_D:]
    w2t = W2.T
    wft = Wf.T
    b1r = b1.reshape(1, _H1)
    b2r = b2.reshape(1, _H2)
    bfr = bf.reshape(1, 1)

    bb = 2048
    grid = (_BATCH // bb,)
    full = lambda i: (0, 0)
    return pl.pallas_call(
        _mlp_body,
        grid=grid,
        in_specs=[
            pl.BlockSpec((bb, 2 * _D), lambda i: (i, 0)),
            pl.BlockSpec((bb, 2 * _D), lambda i: (i, 0)),
            pl.BlockSpec((bb, 1), lambda i: (i, 0)),
            pl.BlockSpec((bb, 1), lambda i: (i, 0)),
            pl.BlockSpec((_D, _H1), full),
            pl.BlockSpec((_D, _H1), full),
            pl.BlockSpec((1, _H1), full),
            pl.BlockSpec((_H1, _H2), full),
            pl.BlockSpec((1, _H2), full),
            pl.BlockSpec((_H2, 1), full),
            pl.BlockSpec((1, 1), full),
        ],
        out_specs=pl.BlockSpec((bb, 1), lambda i: (i, 0)),
        out_shape=jax.ShapeDtypeStruct((_BATCH, 1), jnp.float32),
    )(ue, me, pu, pm, w1u, w1m, b1r, w2t, b2r, wft, bfr)


def kernel(users, movies, U, M, W1, b1, W2, b2, Wf, bf):
    users = users.astype(jnp.int32)
    movies = movies.astype(jnp.int32)
    U2, M2 = _repack(U, M)
    hu = jnp.int32(N_U // 2)
    hm = jnp.int32(N_M // 2)
    ue, me = _sc_gather(users % hu, movies % hm, U2, M2)
    pu = (users >= hu).astype(jnp.float32).reshape(-1, 1)
    pm = (movies >= hm).astype(jnp.float32).reshape(-1, 1)
    return _mlp(ue, me, pu, pm, W1, b1, W2, b2, Wf, bf)


# speedup vs baseline: 1.0708x; 1.0708x over previous
import functools

import jax
import jax.numpy as jnp
from jax import lax
from jax.experimental import pallas as pl
from jax.experimental.pallas import tpu as pltpu
from jax.experimental.pallas import tpu_sc as plsc

_BATCH = 16384
_D = 64
_H1 = 128
_H2 = 64

_IDX_CHUNK = 128


def _sc_gather(users, movies, U, M):
    info = plsc.get_sparse_core_info()
    nw = info.num_cores * info.num_subcores
    b_per_w = _BATCH // nw
    n_chunks = b_per_w // _IDX_CHUNK

    mesh = plsc.VectorSubcoreMesh(core_axis_name="c", subcore_axis_name="s")

    @functools.partial(
        pl.kernel,
        mesh=mesh,
        compiler_params=pltpu.CompilerParams(use_tc_tiling_on_sc=False),
        out_type=[
            jax.ShapeDtypeStruct((_BATCH, _D), jnp.float32),
            jax.ShapeDtypeStruct((_BATCH, _D), jnp.float32),
        ],
        scratch_types=[
            pltpu.VMEM((n_chunks, _IDX_CHUNK), jnp.int32),
            pltpu.VMEM((n_chunks, _IDX_CHUNK), jnp.int32),
            pltpu.VMEM((b_per_w, _D), jnp.float32),
            pltpu.VMEM((b_per_w, _D), jnp.float32),
            pltpu.SemaphoreType.DMA,
            pltpu.SemaphoreType.DMA,
        ],
    )
    def gather_kernel(users_hbm, movies_hbm, u_hbm, m_hbm, ue_hbm, me_hbm,
                      uidx_v, midx_v, urows_v, mrows_v, usem, msem):
        wid = lax.axis_index("s") * info.num_cores + lax.axis_index("c")
        pltpu.sync_copy(users_hbm.at[wid], uidx_v)
        pltpu.sync_copy(movies_hbm.at[wid], midx_v)
        copies = []
        for j in range(n_chunks):
            dst = pl.ds(j * _IDX_CHUNK, _IDX_CHUNK)
            copies.append(pltpu.async_copy(u_hbm.at[uidx_v.at[j]],
                                           urows_v.at[dst], usem))
            copies.append(pltpu.async_copy(m_hbm.at[midx_v.at[j]],
                                           mrows_v.at[dst], msem))
        for c in copies:
            c.wait()
        base = wid * b_per_w
        pltpu.sync_copy(urows_v, ue_hbm.at[pl.ds(base, b_per_w)])
        pltpu.sync_copy(mrows_v, me_hbm.at[pl.ds(base, b_per_w)])

    users3 = users.astype(jnp.int32).reshape(nw, n_chunks, _IDX_CHUNK)
    movies3 = movies.astype(jnp.int32).reshape(nw, n_chunks, _IDX_CHUNK)
    return gather_kernel(users3, movies3, U, M)


def _mlp_body(ue_ref, me_ref, w1u_ref, w1m_ref, b1_ref, w2_ref, b2_ref,
              wf_ref, bf_ref, out_ref):
    x = jnp.dot(ue_ref[...], w1u_ref[...], preferred_element_type=jnp.float32)
    x = x + jnp.dot(me_ref[...], w1m_ref[...],
                    preferred_element_type=jnp.float32)
    x = jnp.maximum(x + b1_ref[...], 0.0)
    x = jnp.dot(x, w2_ref[...], preferred_element_type=jnp.float32)
    x = jnp.maximum(x + b2_ref[...], 0.0)
    x = jnp.dot(x, wf_ref[...], preferred_element_type=jnp.float32)
    out_ref[...] = jax.nn.sigmoid(x + bf_ref[...])


def _mlp(ue, me, W1, b1, W2, b2, Wf, bf):
    w1t = W1.T
    w1u = w1t[:_D]
    w1m = w1t[_D:]
    w2t = W2.T
    wft = Wf.T
    b1r = b1.reshape(1, _H1)
    b2r = b2.reshape(1, _H2)
    bfr = bf.reshape(1, 1)

    bb = 2048
    grid = (_BATCH // bb,)
    full = lambda i: (0, 0)
    return pl.pallas_call(
        _mlp_body,
        grid=grid,
        in_specs=[
            pl.BlockSpec((bb, _D), lambda i: (i, 0)),
            pl.BlockSpec((bb, _D), lambda i: (i, 0)),
            pl.BlockSpec((_D, _H1), full),
            pl.BlockSpec((_D, _H1), full),
            pl.BlockSpec((1, _H1), full),
            pl.BlockSpec((_H1, _H2), full),
            pl.BlockSpec((1, _H2), full),
            pl.BlockSpec((_H2, 1), full),
            pl.BlockSpec((1, 1), full),
        ],
        out_specs=pl.BlockSpec((bb, 1), lambda i: (i, 0)),
        out_shape=jax.ShapeDtypeStruct((_BATCH, 1), jnp.float32),
    )(ue, me, w1u, w1m, b1r, w2t, b2r, wft, bfr)


def kernel(users, movies, U, M, W1, b1, W2, b2, Wf, bf):
    ue, me = _sc_gather(users, movies, U, M)
    return _mlp(ue, me, W1, b1, W2, b2, Wf, bf)
